# baseline (speedup 1.0000x reference)
import math

import jax
import jax.numpy as jnp
from jax import lax
from jax.experimental import pallas as pl
from jax.experimental.pallas import tpu as pltpu
from jax.experimental.pallas import tpu_sc as plsc

B = 4096
CD = 128
NPOS = 1024
HID = 3584
NSLOT = 6
BM = 512
PK = CD // 2
TROWS = 4 * NPOS

NCHK = 2
BC = B // NCHK
NWORK = 32
NSUB = 16
TOTC = NSLOT * BC
RPW = BC // NWORK
BPW = TOTC // NWORK
GCH = 128
NCH = BPW // GCH
LANES = 16


def _make_sc_body(chunk):
    def _sc_body(bbox_hbm, table_hbm, out_hbm,
                 bb_v, idx_v, rows_v, shared_tab, tsem, gsem, osem):
        cid = lax.axis_index("c")
        sid = lax.axis_index("s")
        wid = sid * 2 + cid
        trows = TROWS // NSUB
        tstage = pltpu.async_copy(
            table_hbm.at[pl.ds(sid * trows, trows)],
            shared_tab.at[pl.ds(sid * trows, trows)],
            tsem,
        )
        pltpu.sync_copy(bbox_hbm.at[pl.ds(chunk * BC + wid * RPW, RPW)], bb_v)
        lane = lax.iota(jnp.int32, LANES)
        for blk in range(RPW // LANES):
            rowids = lane + blk * LANES
            coords = []
            for c in range(4):
                v = plsc.load_gather(bb_v, [rowids, jnp.full((LANES,), c, jnp.int32)])
                coords.append(jnp.clip((v * 1023.0).astype(jnp.int32), 0, 1023))
            x0, y0, x1, y1 = coords
            w = jnp.clip(x1 - x0, 0, 1023)
            h = jnp.clip(y1 - y0, 0, 1023)
            pos = rowids * NSLOT
            plsc.store_scatter(idx_v, [pos], x0)
            plsc.store_scatter(idx_v, [pos + 1], y0 + NPOS)
            plsc.store_scatter(idx_v, [pos + 2], x1)
            plsc.store_scatter(idx_v, [pos + 3], y1 + NPOS)
            plsc.store_scatter(idx_v, [pos + 4], w + 2 * NPOS)
            plsc.store_scatter(idx_v, [pos + 5], h + 3 * NPOS)
        tstage.wait()
        plsc.subcore_barrier()
        gcopies = []
        for j in range(NCH):
            gcopies.append(
                pltpu.async_copy(
                    shared_tab.at[idx_v.at[pl.ds(j * GCH, GCH)]],
                    rows_v.at[pl.ds(j * GCH, GCH)],
                    gsem,
                )
            )
        ocopies = []
        for j in range(NCH):
            gcopies[j].wait()
            ocopies.append(
                pltpu.async_copy(
                    rows_v.at[pl.ds(j * GCH, GCH)],
                    out_hbm.at[pl.ds(wid * BPW + j * GCH, GCH)],
                    osem,
                )
            )
        for c in ocopies:
            c.wait()
    return _sc_body


def _sc_gather_chunk(bbox, packed_tables, chunk):
    mesh = plsc.VectorSubcoreMesh(core_axis_name="c", subcore_axis_name="s")
    return pl.kernel(
        _make_sc_body(chunk),
        out_type=jax.ShapeDtypeStruct((TOTC, PK), jnp.float32),
        mesh=mesh,
        scratch_types=[
            pltpu.VMEM((RPW, 4), jnp.float32),
            pltpu.VMEM((BPW,), jnp.int32),
            pltpu.VMEM((BPW, PK), jnp.float32),
            pltpu.VMEM_SHARED((TROWS, PK), jnp.float32),
            pltpu.SemaphoreType.DMA,
            pltpu.SemaphoreType.DMA,
            pltpu.SemaphoreType.DMA,
        ],
        compiler_params=pltpu.CompilerParams(
            use_tc_tiling_on_sc=False, needs_layout_passes=False
        ),
    )(bbox, packed_tables)


def _proj_body(g_ref, wlo_ref, whi_ref, gam_ref, bet_ref, o_ref):
    u = lax.bitcast_convert_type(g_ref[...], jnp.int32)
    lo = lax.bitcast_convert_type(u << 16, jnp.float32).astype(jnp.bfloat16)
    hi = lax.bitcast_convert_type(u & jnp.int32(-65536), jnp.float32).astype(jnp.bfloat16)
    hi = jnp.concatenate([hi, jnp.ones((BM, 1), jnp.bfloat16)], axis=1)
    dn = (((1,), (0,)), ((), ()))
    z = lax.dot_general(lo, wlo_ref[...], dn, preferred_element_type=jnp.float32)
    z = z + lax.dot_general(hi, whi_ref[...], dn, preferred_element_type=jnp.float32)
    mu = jnp.mean(z, axis=1, keepdims=True)
    ms = jnp.mean(z * z, axis=1, keepdims=True)
    inv = lax.rsqrt(ms - mu * mu + 1e-5)
    zn = (z - mu) * inv * gam_ref[...] + bet_ref[...]
    o_ref[...] = zn * 0.5 * (1.0 + lax.erf(zn * (1.0 / math.sqrt(2.0))))


def _proj_chunk0(g, w_lo, w_hi, ln_gamma, ln_beta):
    return pl.pallas_call(
        _proj_body,
        grid=(BC // BM,),
        in_specs=[
            pl.BlockSpec((BM, NSLOT * PK), lambda i: (i, 0)),
            pl.BlockSpec((NSLOT * PK, HID), lambda i: (0, 0)),
            pl.BlockSpec((NSLOT * PK + 1, HID), lambda i: (0, 0)),
            pl.BlockSpec((1, HID), lambda i: (0, 0)),
            pl.BlockSpec((1, HID), lambda i: (0, 0)),
        ],
        out_specs=pl.BlockSpec((BM, HID), lambda i: (i, 0)),
        out_shape=jax.ShapeDtypeStruct((B, HID), jnp.float32),
    )(g, w_lo, w_hi, ln_gamma, ln_beta)


def _proj_body1(g_ref, wlo_ref, whi_ref, gam_ref, bet_ref, prev_ref, o_ref):
    del prev_ref
    _proj_body(g_ref, wlo_ref, whi_ref, gam_ref, bet_ref, o_ref)


def _proj_chunk1(g, w_lo, w_hi, ln_gamma, ln_beta, prev):
    nblk = BC // BM
    return pl.pallas_call(
        _proj_body1,
        grid=(nblk,),
        in_specs=[
            pl.BlockSpec((BM, NSLOT * PK), lambda i: (i, 0)),
            pl.BlockSpec((NSLOT * PK, HID), lambda i: (0, 0)),
            pl.BlockSpec((NSLOT * PK + 1, HID), lambda i: (0, 0)),
            pl.BlockSpec((1, HID), lambda i: (0, 0)),
            pl.BlockSpec((1, HID), lambda i: (0, 0)),
            pl.BlockSpec(memory_space=pl.ANY),
        ],
        out_specs=pl.BlockSpec((BM, HID), lambda i: (i + nblk, 0)),
        out_shape=jax.ShapeDtypeStruct((B, HID), jnp.float32),
        input_output_aliases={5: 0},
    )(g, w_lo, w_hi, ln_gamma, ln_beta, prev)


def kernel(bbox, x_table, y_table, w_table, h_table, proj_W, proj_b, ln_gamma, ln_beta):
    tables = jnp.concatenate(
        [x_table, y_table, w_table, h_table], axis=0
    ).astype(jnp.bfloat16)
    packed_tables = lax.bitcast_convert_type(
        tables.reshape(TROWS, PK, 2), jnp.float32
    )
    w_pair = proj_W.astype(jnp.bfloat16).reshape(NSLOT * PK, 2, HID)
    w_lo = w_pair[:, 0]
    w_hi = jnp.concatenate(
        [w_pair[:, 1], proj_b.astype(jnp.bfloat16).reshape(1, HID)], axis=0
    )
    lg = ln_gamma.reshape(1, HID)
    lb = ln_beta.reshape(1, HID)
    g0 = _sc_gather_chunk(bbox, packed_tables, 0).reshape(BC, NSLOT * PK)
    g1 = _sc_gather_chunk(bbox, packed_tables, 1).reshape(BC, NSLOT * PK)
    o = _proj_chunk0(g0, w_lo, w_hi, lg, lb)
    return _proj_chunk1(g1, w_lo, w_hi, lg, lb, o)
